# baseline (speedup 1.0000x reference)
import functools

import jax
import jax.numpy as jnp
from jax import lax
from jax.experimental import pallas as pl
from jax.experimental.pallas import tpu as pltpu
from jax.experimental.pallas import tpu_sc as plsc

NC, NS = 2, 16
NW = NC * NS
CHUNK = 128
BATCH, SEQ = 4096, 50
TOTAL = BATCH * SEQ
ROWS = TOTAL // CHUNK
CPW = ROWS // NW
DIM = 64

_mesh = plsc.VectorSubcoreMesh(core_axis_name="c", subcore_axis_name="s")


@functools.partial(
    pl.kernel,
    out_type=jax.ShapeDtypeStruct((TOTAL, DIM), jnp.float32),
    mesh=_mesh,
    scratch_types=[
        pltpu.VMEM((CPW, CHUNK), jnp.int32),
        pltpu.VMEM((CHUNK, DIM), jnp.float32),
        pltpu.VMEM((CHUNK, DIM), jnp.float32),
        pltpu.SemaphoreType.DMA,
        pltpu.SemaphoreType.DMA,
    ],
    compiler_params=pltpu.CompilerParams(use_tc_tiling_on_sc=False),
)
def _gather_kernel(idx_hbm, table_hbm, out_hbm, idx_v, buf0, buf1, sem0, sem1):
    wid = lax.axis_index("s") * NC + lax.axis_index("c")
    base = wid * CPW

    pltpu.sync_copy(idx_hbm.at[wid], idx_v)

    def gather_start(j, buf, sem):
        pltpu.async_copy(table_hbm.at[idx_v.at[j]], buf, sem)

    def gather_wait(j, buf, sem):
        pltpu.make_async_copy(table_hbm.at[idx_v.at[j]], buf, sem).wait()

    def write_out(j, buf):
        pltpu.sync_copy(buf, out_hbm.at[pl.ds((base + j) * CHUNK, CHUNK)])

    gather_start(0, buf0, sem0)

    def body(t, carry):
        j0 = 2 * t
        gather_start(j0 + 1, buf1, sem1)
        gather_wait(j0, buf0, sem0)
        write_out(j0, buf0)
        gather_start(j0 + 2, buf0, sem0)
        gather_wait(j0 + 1, buf1, sem1)
        write_out(j0 + 1, buf1)
        return carry

    lax.fori_loop(0, CPW // 2 - 1, body, 0)

    gather_start(CPW - 1, buf1, sem1)
    gather_wait(CPW - 2, buf0, sem0)
    write_out(CPW - 2, buf0)
    gather_wait(CPW - 1, buf1, sem1)
    write_out(CPW - 1, buf1)


def kernel(x, table):
    idx = x.reshape(NW, CPW, CHUNK).astype(jnp.int32)
    flat = _gather_kernel(idx, table)
    return flat.reshape(BATCH, SEQ, DIM)


# speedup vs baseline: 1.0080x; 1.0080x over previous
import functools

import jax
import jax.numpy as jnp
from jax import lax
from jax.experimental import pallas as pl
from jax.experimental.pallas import tpu as pltpu
from jax.experimental.pallas import tpu_sc as plsc

NC, NS = 2, 16
NW = NC * NS
CHUNK = 128
BATCH, SEQ = 4096, 50
TOTAL = BATCH * SEQ
ROWS = TOTAL // CHUNK
CPW = ROWS // NW
DIM = 64

_mesh = plsc.VectorSubcoreMesh(core_axis_name="c", subcore_axis_name="s")


GROUP = 5
NG = CPW // GROUP
GROWS = GROUP * CHUNK


@functools.partial(
    pl.kernel,
    out_type=jax.ShapeDtypeStruct((TOTAL, DIM), jnp.float32),
    mesh=_mesh,
    scratch_types=[
        pltpu.VMEM((CPW, CHUNK), jnp.int32),
        pltpu.VMEM((GROWS, DIM), jnp.float32),
        pltpu.VMEM((GROWS, DIM), jnp.float32),
        pltpu.SemaphoreType.DMA,
        pltpu.SemaphoreType.DMA,
        pltpu.SemaphoreType.DMA,
        pltpu.SemaphoreType.DMA,
    ],
    compiler_params=pltpu.CompilerParams(use_tc_tiling_on_sc=False),
)
def _gather_kernel(idx_hbm, table_hbm, out_hbm, idx_v, buf_a, buf_b,
                   gsem_a, gsem_b, wsem_a, wsem_b):
    wid = lax.axis_index("s") * NC + lax.axis_index("c")
    base = wid * CPW

    pltpu.sync_copy(idx_hbm.at[wid], idx_v)

    def fire(g, buf, gsem):
        for c in range(GROUP):
            pltpu.async_copy(table_hbm.at[idx_v.at[g * GROUP + c]],
                             buf.at[pl.ds(c * CHUNK, CHUNK)], gsem)

    def drain(g, buf, gsem):
        for c in range(GROUP):
            pltpu.make_async_copy(table_hbm.at[idx_v.at[g * GROUP + c]],
                                  buf.at[pl.ds(c * CHUNK, CHUNK)], gsem).wait()

    def wstart(g, buf, wsem):
        pltpu.async_copy(
            buf, out_hbm.at[pl.ds((base + g * GROUP) * CHUNK, GROWS)], wsem)

    def wwait(g, buf, wsem):
        pltpu.make_async_copy(
            buf, out_hbm.at[pl.ds((base + g * GROUP) * CHUNK, GROWS)],
            wsem).wait()

    fire(0, buf_a, gsem_a)
    fire(1, buf_b, gsem_b)

    def body(t, carry):
        g = 2 * t
        drain(g, buf_a, gsem_a)
        wstart(g, buf_a, wsem_a)
        wwait(g, buf_a, wsem_a)
        fire(g + 2, buf_a, gsem_a)
        drain(g + 1, buf_b, gsem_b)
        wstart(g + 1, buf_b, wsem_b)
        wwait(g + 1, buf_b, wsem_b)
        fire(g + 3, buf_b, gsem_b)
        return carry

    lax.fori_loop(0, NG // 2 - 1, body, 0)

    drain(NG - 2, buf_a, gsem_a)
    wstart(NG - 2, buf_a, wsem_a)
    drain(NG - 1, buf_b, gsem_b)
    wstart(NG - 1, buf_b, wsem_b)
    wwait(NG - 2, buf_a, wsem_a)
    wwait(NG - 1, buf_b, wsem_b)


def kernel(x, table):
    idx = x.reshape(NW, CPW, CHUNK).astype(jnp.int32)
    flat = _gather_kernel(idx, table)
    return flat.reshape(BATCH, SEQ, DIM)
